# baseline (speedup 1.0000x reference)
import functools

import jax
import jax.numpy as jnp
from jax import lax
from jax.experimental import pallas as pl
from jax.experimental.pallas import tpu as pltpu
from jax.experimental.pallas import tpu_sc as plsc

B, C, H, W = 4, 192, 224, 224
PLANE = H * W
S = 8
D = PLANE // S
ROWS = B * C * S
NC, NS = 2, 16
NW = NC * NS
PER_W = ROWS // NW
G = 8
NG = PER_W // G

_MESH = plsc.VectorSubcoreMesh(core_axis_name="c", subcore_axis_name="s")


@functools.partial(
    pl.kernel,
    out_type=jax.ShapeDtypeStruct((ROWS, D), jnp.float32),
    mesh=_MESH,
    scratch_types=[
        pltpu.VMEM((NG, G), jnp.int32),
        pltpu.VMEM((G, D), jnp.float32),
        pltpu.VMEM((G, D), jnp.float32),
        pltpu.SemaphoreType.DMA,
        pltpu.SemaphoreType.DMA,
        pltpu.SemaphoreType.DMA,
        pltpu.SemaphoreType.DMA,
    ],
)
def _permute_rows(in_hbm, idx_hbm, out_hbm, idx_v, buf0, buf1, g0, g1, s0, s1):
    wid = lax.axis_index("s") * NC + lax.axis_index("c")
    base = wid * PER_W
    pltpu.sync_copy(idx_hbm.at[wid], idx_v)
    bufs = (buf0, buf1)
    gsem = (g0, g1)
    ssem = (s0, s1)

    for b in range(2):
        pltpu.async_copy(in_hbm.at[idx_v.at[b]], bufs[b], gsem[b])

    for g in range(NG):
        b = g & 1
        pltpu.make_async_copy(in_hbm.at[idx_v.at[g]], bufs[b], gsem[b]).wait()
        pltpu.async_copy(bufs[b], out_hbm.at[pl.ds(base + g * G, G)], ssem[b])
        if g + 2 < NG:
            pltpu.make_async_copy(
                bufs[b], out_hbm.at[pl.ds(base + g * G, G)], ssem[b]).wait()
            pltpu.async_copy(in_hbm.at[idx_v.at[g + 2]], bufs[b], gsem[b])

    for g in (NG - 2, NG - 1):
        b = g & 1
        pltpu.make_async_copy(
            bufs[b], out_hbm.at[pl.ds(base + g * G, G)], ssem[b]).wait()


def kernel(input, indices):
    src_plane = (jnp.arange(B, dtype=jnp.int32)[:, None] * C
                 + indices[None, :].astype(jnp.int32)).reshape(-1)
    src_chunk = (src_plane[:, None] * S
                 + jnp.arange(S, dtype=jnp.int32)[None, :])
    idx = src_chunk.reshape(NW, NG, G)
    out = _permute_rows(input.reshape(ROWS, D), idx)
    return out.reshape(input.shape), 0.0


# speedup vs baseline: 1.1098x; 1.1098x over previous
import functools

import jax
import jax.numpy as jnp
from jax import lax
from jax.experimental import pallas as pl
from jax.experimental.pallas import tpu as pltpu
from jax.experimental.pallas import tpu_sc as plsc

B, C, H, W = 4, 192, 224, 224
PLANE = H * W
S = 1
D = PLANE // S
ROWS = B * C * S
NC, NS = 2, 16
NW = NC * NS
PER_W = ROWS // NW
G = 1
NG = PER_W // G

_MESH = plsc.VectorSubcoreMesh(core_axis_name="c", subcore_axis_name="s")


@functools.partial(
    pl.kernel,
    out_type=jax.ShapeDtypeStruct((ROWS, D), jnp.float32),
    mesh=_MESH,
    scratch_types=[
        pltpu.VMEM((NG, G), jnp.int32),
        pltpu.VMEM((G, D), jnp.float32),
        pltpu.VMEM((G, D), jnp.float32),
        pltpu.SemaphoreType.DMA,
        pltpu.SemaphoreType.DMA,
        pltpu.SemaphoreType.DMA,
        pltpu.SemaphoreType.DMA,
    ],
)
def _permute_rows(in_hbm, idx_hbm, out_hbm, idx_v, buf0, buf1, g0, g1, s0, s1):
    wid = lax.axis_index("s") * NC + lax.axis_index("c")
    base = wid * PER_W
    pltpu.sync_copy(idx_hbm.at[wid], idx_v)
    bufs = (buf0, buf1)
    gsem = (g0, g1)
    ssem = (s0, s1)

    for b in range(2):
        pltpu.async_copy(in_hbm.at[idx_v.at[b]], bufs[b], gsem[b])

    for g in range(NG):
        b = g & 1
        pltpu.make_async_copy(in_hbm.at[idx_v.at[g]], bufs[b], gsem[b]).wait()
        pltpu.async_copy(bufs[b], out_hbm.at[pl.ds(base + g * G, G)], ssem[b])
        if g + 2 < NG:
            pltpu.make_async_copy(
                bufs[b], out_hbm.at[pl.ds(base + g * G, G)], ssem[b]).wait()
            pltpu.async_copy(in_hbm.at[idx_v.at[g + 2]], bufs[b], gsem[b])

    for g in (NG - 2, NG - 1):
        b = g & 1
        pltpu.make_async_copy(
            bufs[b], out_hbm.at[pl.ds(base + g * G, G)], ssem[b]).wait()


def kernel(input, indices):
    src_plane = (jnp.arange(B, dtype=jnp.int32)[:, None] * C
                 + indices[None, :].astype(jnp.int32)).reshape(-1)
    src_chunk = (src_plane[:, None] * S
                 + jnp.arange(S, dtype=jnp.int32)[None, :])
    idx = src_chunk.reshape(NW, NG, G)
    out = _permute_rows(input.reshape(ROWS, D), idx)
    return out.reshape(input.shape), 0.0


# speedup vs baseline: 3.4852x; 3.1402x over previous
import functools

import jax
import jax.numpy as jnp
from jax import lax
from jax.experimental import pallas as pl
from jax.experimental.pallas import tpu as pltpu
from jax.experimental.pallas import tpu_sc as plsc

B, C, H, W = 4, 192, 224, 224
PLANES = B * C
NC, NS = 2, 16
NW = NC * NS
NG = PLANES // NW

_MESH = plsc.VectorSubcoreMesh(core_axis_name="c", subcore_axis_name="s")


@functools.partial(
    pl.kernel,
    out_type=jax.ShapeDtypeStruct((PLANES, H, W), jnp.float32),
    mesh=_MESH,
    scratch_types=[
        pltpu.VMEM((32,), jnp.int32),
        pltpu.VMEM((1, H, W), jnp.float32),
        pltpu.VMEM((1, H, W), jnp.float32),
        pltpu.SemaphoreType.DMA,
        pltpu.SemaphoreType.DMA,
        pltpu.SemaphoreType.DMA,
        pltpu.SemaphoreType.DMA,
    ],
)
def _permute_planes(in_hbm, idx_hbm, out_hbm, idx_v, buf0, buf1, g0, g1, s0, s1):
    wid = lax.axis_index("s") * NC + lax.axis_index("c")
    base = wid * NG
    pltpu.sync_copy(idx_hbm.at[wid], idx_v)
    bufs = (buf0, buf1)
    gsem = (g0, g1)
    ssem = (s0, s1)

    lo, hi = idx_v[pl.ds(0, 16)], idx_v[pl.ds(16, 16)]

    def src(g):
        return lo[g] if g < 16 else hi[g - 16]

    for b in range(2):
        pltpu.async_copy(in_hbm.at[pl.ds(src(b), 1)], bufs[b], gsem[b])

    for g in range(NG):
        b = g & 1
        pltpu.make_async_copy(
            in_hbm.at[pl.ds(0, 1)], bufs[b], gsem[b]).wait()
        pltpu.async_copy(bufs[b], out_hbm.at[pl.ds(base + g, 1)], ssem[b])
        if g + 2 < NG:
            pltpu.make_async_copy(
                bufs[b], out_hbm.at[pl.ds(base + g, 1)], ssem[b]).wait()
            pltpu.async_copy(
                in_hbm.at[pl.ds(src(g + 2), 1)], bufs[b], gsem[b])

    for g in (NG - 2, NG - 1):
        b = g & 1
        pltpu.make_async_copy(
            bufs[b], out_hbm.at[pl.ds(base + g, 1)], ssem[b]).wait()


def kernel(input, indices):
    src_plane = (jnp.arange(B, dtype=jnp.int32)[:, None] * C
                 + indices[None, :].astype(jnp.int32))
    idx = jnp.pad(src_plane.reshape(NW, NG), ((0, 0), (0, 32 - NG)))
    out = _permute_planes(input.reshape(PLANES, H, W), idx)
    return out.reshape(input.shape), 0.0
